# baseline (speedup 1.0000x reference)
import functools

import jax
import jax.numpy as jnp
from jax import lax
from jax.experimental import pallas as pl
from jax.experimental.pallas import tpu as pltpu
from jax.experimental.pallas import tpu_sc as plsc

N = 10000
E = 320000
D = 128
L = 10

NS = 16
EPW = E // (2 * NS)
C = 80
SB = 5
NSC = EPW // (SB * C)
RPT = 624
REM = N - NS * RPT


def _sc_aggregate(hl, src_r, dst_r, w_r):
    mesh = plsc.VectorSubcoreMesh(core_axis_name="c", subcore_axis_name="s",
                                  num_cores=1)

    @functools.partial(
        pl.kernel,
        mesh=mesh,
        out_type=jax.ShapeDtypeStruct((N, D), jnp.float32),
        scratch_types=[
            pltpu.VMEM((SB, C), jnp.int32),
            pltpu.VMEM((SB, C), jnp.int32),
            pltpu.VMEM((SB, C), jnp.float32),
            pltpu.VMEM((3, C, D), jnp.float32),
            pltpu.VMEM_SHARED((N, D), jnp.float32),
            pltpu.SemaphoreType.DMA,
            pltpu.SemaphoreType.DMA,
        ],
    )
    def k(hl_hbm, src_hbm, dst_hbm, w_hbm, out_hbm,
          src_v, dst_v, w_v, rows_v, acc, gsem, ssem):
        s = lax.axis_index("s")

        def zrow_body(r, carry):
            for kk in range(D // 16):
                rows_v[0, r, pl.ds(kk * 16, 16)] = (
                    jnp.zeros((16,), jnp.float32))
            return carry
        lax.fori_loop(0, C, zrow_body, 0)
        for t in range(RPT // C):
            pltpu.sync_copy(rows_v.at[0],
                            acc.at[pl.ds(s * RPT + t * C, C)])
        pltpu.sync_copy(rows_v.at[0, pl.ds(0, RPT % C)],
                        acc.at[pl.ds(s * RPT + (RPT // C) * C, RPT % C)])

        @pl.when(s == 0)
        def _zero_rem():
            pltpu.sync_copy(rows_v.at[0, pl.ds(0, REM)],
                            acc.at[pl.ds(NS * RPT, REM)])
        plsc.subcore_barrier()

        def scale_chunk(b, j):
            def group_body(g, cc):
                w16 = w_v[j, pl.ds(g * 16, 16)]
                for e in range(16):
                    ws = w16[e]
                    r = g * 16 + e
                    for kk in range(D // 16):
                        sl = pl.ds(kk * 16, 16)
                        rows_v[b, r, sl] = rows_v[b, r, sl] * ws
                return cc
            lax.fori_loop(0, C // 16, group_body, 0)

        def super_body(t, carry):
            pltpu.sync_copy(src_hbm.at[s, t], src_v)
            pltpu.sync_copy(dst_hbm.at[s, t], dst_v)
            pltpu.sync_copy(w_hbm.at[s, t], w_v)

            h = [None] * SB
            sc = [None] * SB
            h[0] = pltpu.async_copy(hl_hbm.at[src_v.at[0]], rows_v.at[0],
                                    gsem)
            h[1] = pltpu.async_copy(hl_hbm.at[src_v.at[1]], rows_v.at[1],
                                    gsem)
            for j in range(SB):
                b = j % 3
                if j + 2 < SB:
                    if j >= 1:
                        sc[j - 1].wait()
                    h[j + 2] = pltpu.async_copy(
                        hl_hbm.at[src_v.at[j + 2]], rows_v.at[(j + 2) % 3],
                        gsem)
                h[j].wait()
                scale_chunk(b, j)
                sc[j] = pltpu.async_copy(rows_v.at[b], acc.at[dst_v.at[j]],
                                         ssem, add=True)
            sc[SB - 3].wait()
            sc[SB - 2].wait()
            sc[SB - 1].wait()
            return carry
        lax.fori_loop(0, NSC, super_body, 0)

        plsc.subcore_barrier()
        pltpu.sync_copy(acc.at[pl.ds(s * RPT, RPT)],
                        out_hbm.at[pl.ds(s * RPT, RPT)])

        @pl.when(s == 0)
        def _write_rem():
            pltpu.sync_copy(acc.at[pl.ds(NS * RPT, REM)],
                            out_hbm.at[pl.ds(NS * RPT, REM)])

    return k(hl, src_r, dst_r, w_r)


_RB = 1000
_GRID = N // _RB


def _tc_first(x, w0t, b0, t0):
    def body(x_ref, w_ref, b_ref, t_ref, hl_ref, hid_ref):
        xv = x_ref[...]
        hid_ref[...] = t_ref[0, 0] * xv
        hl_ref[...] = (jnp.dot(xv, w_ref[...],
                               preferred_element_type=jnp.float32)
                       + b_ref[...])
    return pl.pallas_call(
        body,
        grid=(_GRID,),
        in_specs=[
            pl.BlockSpec((_RB, D), lambda i: (i, 0)),
            pl.BlockSpec((D, D), lambda i: (0, 0)),
            pl.BlockSpec((1, D), lambda i: (0, 0)),
            pl.BlockSpec((1, 1), lambda i: (0, 0)),
        ],
        out_specs=[
            pl.BlockSpec((_RB, D), lambda i: (i, 0)),
            pl.BlockSpec((_RB, D), lambda i: (i, 0)),
        ],
        out_shape=[
            jax.ShapeDtypeStruct((N, D), jnp.float32),
            jax.ShapeDtypeStruct((N, D), jnp.float32),
        ],
    )(x, w0t, b0, t0)


def _tc_mid(p0, p1, hidden, wt, bvec, t):
    def body(p0_ref, p1_ref, hid_ref, w_ref, b_ref, t_ref, hl_ref, hido_ref):
        h = jnp.maximum(p0_ref[...] + p1_ref[...], 0.0)
        hido_ref[...] = hid_ref[...] + t_ref[0, 0] * h
        hl_ref[...] = (jnp.dot(h, w_ref[...],
                               preferred_element_type=jnp.float32)
                       + b_ref[...])
    return pl.pallas_call(
        body,
        grid=(_GRID,),
        in_specs=[
            pl.BlockSpec((_RB, D), lambda i: (i, 0)),
            pl.BlockSpec((_RB, D), lambda i: (i, 0)),
            pl.BlockSpec((_RB, D), lambda i: (i, 0)),
            pl.BlockSpec((D, D), lambda i: (0, 0)),
            pl.BlockSpec((1, D), lambda i: (0, 0)),
            pl.BlockSpec((1, 1), lambda i: (0, 0)),
        ],
        out_specs=[
            pl.BlockSpec((_RB, D), lambda i: (i, 0)),
            pl.BlockSpec((_RB, D), lambda i: (i, 0)),
        ],
        out_shape=[
            jax.ShapeDtypeStruct((N, D), jnp.float32),
            jax.ShapeDtypeStruct((N, D), jnp.float32),
        ],
    )(p0, p1, hidden, wt, bvec, t)


def _tc_last(p0, p1, hidden, t):
    def body(p0_ref, p1_ref, hid_ref, t_ref, hido_ref):
        hido_ref[...] = hid_ref[...] + t_ref[0, 0] * jnp.maximum(
            p0_ref[...] + p1_ref[...], 0.0)
    return pl.pallas_call(
        body,
        grid=(_GRID,),
        in_specs=[
            pl.BlockSpec((_RB, D), lambda i: (i, 0)),
            pl.BlockSpec((_RB, D), lambda i: (i, 0)),
            pl.BlockSpec((_RB, D), lambda i: (i, 0)),
            pl.BlockSpec((1, 1), lambda i: (0, 0)),
        ],
        out_specs=pl.BlockSpec((_RB, D), lambda i: (i, 0)),
        out_shape=jax.ShapeDtypeStruct((N, D), jnp.float32),
    )(p0, p1, hidden, t)


def kernel(x, edge_index, edge_weight, W, b, temp):
    src_r = edge_index[0].reshape(2, NS, NSC, SB, C)
    dst_r = edge_index[1].reshape(2, NS, NSC, SB, C)
    w_r = edge_weight.reshape(2, NS, NSC, SB, C)
    wt = jnp.swapaxes(W, 1, 2)
    b2 = b.reshape(L, 1, D)
    tc = temp.reshape(L + 1, 1, 1)

    hl, hidden = _tc_first(x, wt[0], b2[0], tc[0])
    for i in range(1, L):
        p0 = _sc_aggregate(hl, src_r[0], dst_r[0], w_r[0])
        p1 = _sc_aggregate(hl, src_r[1], dst_r[1], w_r[1])
        hl, hidden = _tc_mid(p0, p1, hidden, wt[i], b2[i], tc[i])
    p0 = _sc_aggregate(hl, src_r[0], dst_r[0], w_r[0])
    p1 = _sc_aggregate(hl, src_r[1], dst_r[1], w_r[1])
    return _tc_last(p0, p1, hidden, tc[L])
